# Initial kernel scaffold; baseline (speedup 1.0000x reference)
import jax
import jax.numpy as jnp
from jax.experimental import pallas as pl


def kernel(preds_0, preds_1, preds_2, targets, image_size):
    raise NotImplementedError("write your pallas kernel here")


# speedup vs baseline: 1.1110x; 1.1110x over previous
import functools
from math import prod

import jax
import jax.numpy as jnp
import numpy as np
from jax import lax
from jax.experimental import pallas as pl
from jax.experimental.pallas import tpu as pltpu

_INTERPRET = False

_ANCHORS = np.array(
    [10, 13, 16, 30, 33, 23, 30, 61, 62, 45, 59, 119, 116, 90, 156, 198, 373, 326],
    dtype=np.float32,
).reshape(3, 3, 2)
_BAL = (4.0, 1.0, 0.4)
_OFF = ((0.0, 0.0), (1.0, 0.0), (0.0, 1.0), (-1.0, 0.0), (0.0, -1.0))
_GRIDS = ((80, 80), (40, 40), (20, 20))
_B = 16
_N = 256
_NCAND = 3 * 5 * _N
_EPS = 1e-7


def _softplus0(x):
    return jnp.maximum(x, 0.0) + jnp.log1p(jnp.exp(-jnp.abs(x)))


_ATAN_C = (9.9999999755e-01, -3.3333282296e-01, 1.9998230640e-01,
           -1.4261573680e-01, 1.0940198965e-01, -8.3720639484e-02,
           5.7463557856e-02, -3.0717508912e-02, 1.0680719451e-02,
           -1.7437011450e-03)


def _atan_pos(z):
    big = z > 1.0
    t = jnp.where(big, 1.0 / z, z)
    u = t * t
    p = jnp.full_like(u, _ATAN_C[-1])
    for c in _ATAN_C[-2::-1]:
        p = p * u + c
    at = t * p
    return jnp.where(big, (np.pi / 2) - at, at)


def _tc_body(p0_ref, p1_ref, p2_ref, ps0_ref, ps1_ref, ps2_ref, tt_ref,
             img_ref, out_ref, acc_ref):
    g = pl.program_id(0)

    s0 = jnp.sum(_softplus0(p0_ref[0, 0]))
    s1 = jnp.sum(_softplus0(p1_ref[0, 0]))
    s2 = jnp.sum(_softplus0(p2_ref[0, 0]))

    @pl.when(g == 0)
    def _init():
        acc_ref[0] = s0
        acc_ref[1] = s1
        acc_ref[2] = s2

    @pl.when(g != 0)
    def _accum():
        acc_ref[0] += s0
        acc_ref[1] += s1
        acc_ref[2] += s2

    @pl.when(g == 0)
    def _entries():
        imgw = img_ref[0, 0]
        imgh = img_ref[0, 1]
        nb = tt_ref[0:1, :]
        cls_ = tt_ref[1:2, :]
        for lvl, (H, W) in enumerate(_GRIDS):
            ps_ref = (ps0_ref, ps1_ref, ps2_ref)[lvl]
            sx = imgw / W
            sy = imgh / H
            cx = tt_ref[2:3, :] / sx
            cy = tt_ref[3:4, :] / sy
            gw = tt_ref[4:5, :] / sx
            gh = tt_ref[5:6, :] / sy
            cidx = jnp.clip(cls_.astype(jnp.int32) - 1, 0, 79)
            oneh = (lax.broadcasted_iota(jnp.int32, (80, _N), 0)
                    == cidx).astype(jnp.float32)
            box_s = 0.0
            corr_s = 0.0
            cls_s = 0.0
            nv_s = 0.0
            for a in range(3):
                aw = _ANCHORS[lvl, a, 0] / sx
                ah = _ANCHORS[lvl, a, 1] / sy
                rw = gw / aw
                rh = gh / ah
                j2 = (jnp.maximum(jnp.maximum(rw, 1.0 / rw),
                                  jnp.maximum(rh, 1.0 / rh)) < 4.0)
                for o in range(5):
                    xo, yo = _OFF[o]
                    gxf = cx - xo
                    gyf = cy - yo
                    j1 = (gxf >= 0) & (gxf < W) & (gyf >= 0) & (gyf < H)
                    mf = jnp.where(j1 & j2, 1.0, 0.0)
                    gxi = gxf.astype(jnp.int32).astype(jnp.float32)
                    gyi = gyf.astype(jnp.int32).astype(jnp.float32)
                    base = (a * 5 + o) * _N
                    psc = ps_ref[:, base:base + _N]
                    px = 3.0 * jax.nn.sigmoid(psc[0:1, :]) - 1.0
                    py = 3.0 * jax.nn.sigmoid(psc[1:2, :]) - 1.0
                    sw = jax.nn.sigmoid(psc[2:3, :])
                    sh = jax.nn.sigmoid(psc[3:4, :])
                    pw = 4.0 * sw * sw * aw
                    ph = 4.0 * sh * sh * ah
                    tbx = cx - gxi
                    tby = cy - gyi
                    b1x1 = px - pw * 0.5
                    b1x2 = px + pw * 0.5
                    b1y1 = py - ph * 0.5
                    b1y2 = py + ph * 0.5
                    b2x1 = tbx - gw * 0.5
                    b2x2 = tbx + gw * 0.5
                    b2y1 = tby - gh * 0.5
                    b2y2 = tby + gh * 0.5
                    inter = (jnp.clip(jnp.minimum(b1x2, b2x2)
                                      - jnp.maximum(b1x1, b2x1), 0.0)
                             * jnp.clip(jnp.minimum(b1y2, b2y2)
                                        - jnp.maximum(b1y1, b2y1), 0.0))
                    union = pw * ph + gw * gh - inter + _EPS
                    iou = inter / union
                    cw = jnp.maximum(b1x2, b2x2) - jnp.minimum(b1x1, b2x1)
                    chh = jnp.maximum(b1y2, b2y2) - jnp.minimum(b1y1, b2y1)
                    c2 = cw * cw + chh * chh + _EPS
                    rho2 = ((b2x1 + b2x2 - b1x1 - b1x2) ** 2
                            + (b2y1 + b2y2 - b1y1 - b1y2) ** 2) * 0.25
                    v = ((4.0 / np.pi ** 2)
                         * (_atan_pos(gw / (gh + _EPS))
                            - _atan_pos(pw / (ph + _EPS))) ** 2)
                    alpha = v / (v - iou + (1.0 + _EPS))
                    ciou = iou - (rho2 / c2 + v * alpha)
                    box_s += jnp.sum((1.0 - ciou) * mf)
                    corr_s += jnp.sum(psc[4:5, :] * jnp.clip(ciou, 0.0) * mf)
                    nv_s += jnp.sum(mf)
                    xl = psc[5:85, :]
                    bce = (jnp.maximum(xl, 0.0) - xl * oneh
                           + jnp.log1p(jnp.exp(-jnp.abs(xl))))
                    cls_s += jnp.sum(bce * mf)
            acc_ref[3 + lvl] = box_s
            acc_ref[6 + lvl] = corr_s
            acc_ref[9 + lvl] = cls_s
            acc_ref[12 + lvl] = nv_s

    @pl.when(g == pl.num_programs(0) - 1)
    def _final():
        lbox = 0.0
        lobj = 0.0
        lcls = 0.0
        for lvl, (H, W) in enumerate(_GRIDS):
            denom = jnp.maximum(acc_ref[12 + lvl], 1.0)
            lbox += acc_ref[3 + lvl] / denom
            lcls += acc_ref[9 + lvl] / (denom * 80.0)
            lobj += ((acc_ref[lvl] - acc_ref[6 + lvl])
                     / (_B * 3 * H * W)) * _BAL[lvl]
        lbox = lbox * 3.54
        lobj = lobj * 64.3
        lcls = lcls * 37.4
        loss = lbox + lobj + lcls
        out_ref[0] = loss
        out_ref[1] = lbox
        out_ref[2] = lobj
        out_ref[3] = lcls


def _tc_main(preds_0, preds_1, preds_2, ps0, ps1, ps2, tt, img):
    grid = (_B * 3,)

    def pmap(l):
        return lambda g: (g // 3, 85 * (g % 3) + 4, 0, 0)

    out = pl.pallas_call(
        _tc_body,
        grid=grid,
        in_specs=[
            pl.BlockSpec((1, 1, 80, 80), pmap(0)),
            pl.BlockSpec((1, 1, 40, 40), pmap(1)),
            pl.BlockSpec((1, 1, 20, 20), pmap(2)),
            pl.BlockSpec((85, _NCAND), lambda g: (0, 0)),
            pl.BlockSpec((85, _NCAND), lambda g: (0, 0)),
            pl.BlockSpec((85, _NCAND), lambda g: (0, 0)),
            pl.BlockSpec((6, _N), lambda g: (0, 0)),
            pl.BlockSpec(memory_space=pltpu.SMEM),
        ],
        out_specs=pl.BlockSpec(memory_space=pltpu.SMEM),
        out_shape=jax.ShapeDtypeStruct((4,), jnp.float32),
        scratch_shapes=[pltpu.SMEM((16,), jnp.float32)],
        interpret=_INTERPRET,
    )(preds_0, preds_1, preds_2, ps0, ps1, ps2, tt, img)
    return out


def _jax_gather_ps(preds, tt, lvl, imgw, imgh):
    H, W = _GRIDS[lvl]
    sx = imgw / W
    sy = imgh / H
    nb = tt[0].astype(jnp.int32)
    cx = tt[2] / sx
    cy = tt[3] / sy
    flat = preds.reshape(-1)
    cols = []
    for a in range(3):
        for o in range(5):
            xo, yo = _OFF[o]
            gi = jnp.clip((cx - xo).astype(jnp.int32), 0, W - 1)
            gj = jnp.clip((cy - yo).astype(jnp.int32), 0, H - 1)
            b = jnp.clip(nb, 0, _B - 1)
            base = ((b * 255 + 85 * a) * H + gj) * W + gi
            addr = base[None, :] + (jnp.arange(85, dtype=jnp.int32)[:, None]
                                    * (H * W))
            cols.append(jnp.take(flat, addr.reshape(-1)).reshape(85, _N))
    return jnp.concatenate(cols, axis=1)


def kernel(preds_0, preds_1, preds_2, targets, image_size):
    tt = targets[0].T.astype(jnp.float32)
    img = image_size.reshape(1, 2).astype(jnp.float32)
    imgw = image_size[0]
    imgh = image_size[1]
    ps0 = _jax_gather_ps(preds_0, tt, 0, imgw, imgh)
    ps1 = _jax_gather_ps(preds_1, tt, 1, imgw, imgh)
    ps2 = _jax_gather_ps(preds_2, tt, 2, imgw, imgh)
    out = _tc_main(preds_0, preds_1, preds_2, ps0, ps1, ps2, tt, img)
    return (out[0:1], out[1:2], out[2:3], out[3:4])


# speedup vs baseline: 1.6413x; 1.4773x over previous
import functools

import jax
import jax.numpy as jnp
import numpy as np
from jax import lax
from jax.experimental import pallas as pl
from jax.experimental.pallas import tpu as pltpu
from jax.experimental.pallas import tpu_sc as plsc

_INTERPRET = False

_ANCHORS = np.array(
    [10, 13, 16, 30, 33, 23, 30, 61, 62, 45, 59, 119, 116, 90, 156, 198, 373, 326],
    dtype=np.float32,
).reshape(3, 3, 2)
_BAL = (4.0, 1.0, 0.4)
_OFF = ((0.0, 0.0), (1.0, 0.0), (0.0, 1.0), (-1.0, 0.0), (0.0, -1.0))
_GRIDS = ((80, 80), (40, 40), (20, 20))
_B = 16
_N = 256
_NCAND = 3 * 5 * _N
_EPS = 1e-7


def _softplus0(x):
    return jnp.maximum(x, 0.0) + jnp.log1p(jnp.exp(-jnp.abs(x)))


_ATAN_C = (9.9999999755e-01, -3.3333282296e-01, 1.9998230640e-01,
           -1.4261573680e-01, 1.0940198965e-01, -8.3720639484e-02,
           5.7463557856e-02, -3.0717508912e-02, 1.0680719451e-02,
           -1.7437011450e-03)


def _atan_pos(z):
    big = z > 1.0
    t = jnp.where(big, 1.0 / z, z)
    u = t * t
    p = jnp.full_like(u, _ATAN_C[-1])
    for c in _ATAN_C[-2::-1]:
        p = p * u + c
    at = t * p
    return jnp.where(big, (np.pi / 2) - at, at)


def _tc_body(p0_ref, p1_ref, p2_ref, ps0_ref, ps1_ref, ps2_ref, tt_ref,
             img_ref, out_ref, acc_ref):
    g = pl.program_id(0)

    s0 = jnp.sum(_softplus0(p0_ref[0, 0]))
    s1 = jnp.sum(_softplus0(p1_ref[0, 0]))
    s2 = jnp.sum(_softplus0(p2_ref[0, 0]))

    @pl.when(g == 0)
    def _init():
        acc_ref[0] = s0
        acc_ref[1] = s1
        acc_ref[2] = s2

    @pl.when(g != 0)
    def _accum():
        acc_ref[0] += s0
        acc_ref[1] += s1
        acc_ref[2] += s2

    @pl.when(g == 0)
    def _entries():
        imgw = img_ref[0, 0]
        imgh = img_ref[0, 1]
        nb = tt_ref[0:1, :]
        cls_ = tt_ref[1:2, :]
        for lvl, (H, W) in enumerate(_GRIDS):
            ps_ref = (ps0_ref, ps1_ref, ps2_ref)[lvl]
            sx = imgw / W
            sy = imgh / H
            cx = tt_ref[2:3, :] / sx
            cy = tt_ref[3:4, :] / sy
            gw = tt_ref[4:5, :] / sx
            gh = tt_ref[5:6, :] / sy
            cidx = jnp.clip(cls_.astype(jnp.int32) - 1, 0, 79)
            oneh = (lax.broadcasted_iota(jnp.int32, (80, _N), 0)
                    == cidx).astype(jnp.float32)
            box_s = 0.0
            corr_s = 0.0
            cls_s = 0.0
            nv_s = 0.0
            for a in range(3):
                aw = _ANCHORS[lvl, a, 0] / sx
                ah = _ANCHORS[lvl, a, 1] / sy
                rw = gw / aw
                rh = gh / ah
                j2 = (jnp.maximum(jnp.maximum(rw, 1.0 / rw),
                                  jnp.maximum(rh, 1.0 / rh)) < 4.0)
                for o in range(5):
                    xo, yo = _OFF[o]
                    gxf_f = cx - xo
                    gyf_f = cy - yo
                    j1 = (gxf_f >= 0) & (gxf_f < W) & (gyf_f >= 0) & (gyf_f < H)
                    mf_f = jnp.where(j1 & j2, 1.0, 0.0)
                    gxi_f = gxf_f.astype(jnp.int32).astype(jnp.float32)
                    gyi_f = gyf_f.astype(jnp.int32).astype(jnp.float32)
                    for h in range(2):
                        sl = slice(128 * h, 128 * h + 128)
                        mf = mf_f[:, sl]
                        cxh = cx[:, sl]
                        cyh = cy[:, sl]
                        gwh_ = gw[:, sl]
                        ghh = gh[:, sl]
                        psc = ps_ref[(a * 5 + o) * 2 + h]
                        px = 3.0 * jax.nn.sigmoid(psc[0:1, :]) - 1.0
                        py = 3.0 * jax.nn.sigmoid(psc[1:2, :]) - 1.0
                        sw = jax.nn.sigmoid(psc[2:3, :])
                        sh = jax.nn.sigmoid(psc[3:4, :])
                        pw = 4.0 * sw * sw * aw
                        ph = 4.0 * sh * sh * ah
                        tbx = cxh - gxi_f[:, sl]
                        tby = cyh - gyi_f[:, sl]
                        b1x1 = px - pw * 0.5
                        b1x2 = px + pw * 0.5
                        b1y1 = py - ph * 0.5
                        b1y2 = py + ph * 0.5
                        b2x1 = tbx - gwh_ * 0.5
                        b2x2 = tbx + gwh_ * 0.5
                        b2y1 = tby - ghh * 0.5
                        b2y2 = tby + ghh * 0.5
                        inter = (jnp.clip(jnp.minimum(b1x2, b2x2)
                                          - jnp.maximum(b1x1, b2x1), 0.0)
                                 * jnp.clip(jnp.minimum(b1y2, b2y2)
                                            - jnp.maximum(b1y1, b2y1), 0.0))
                        union = pw * ph + gwh_ * ghh - inter + _EPS
                        iou = inter / union
                        cw = jnp.maximum(b1x2, b2x2) - jnp.minimum(b1x1, b2x1)
                        chh = jnp.maximum(b1y2, b2y2) - jnp.minimum(b1y1, b2y1)
                        c2 = cw * cw + chh * chh + _EPS
                        rho2 = ((b2x1 + b2x2 - b1x1 - b1x2) ** 2
                                + (b2y1 + b2y2 - b1y1 - b1y2) ** 2) * 0.25
                        v = ((4.0 / np.pi ** 2)
                             * (_atan_pos(gwh_ / (ghh + _EPS))
                                - _atan_pos(pw / (ph + _EPS))) ** 2)
                        alpha = v / (v - iou + (1.0 + _EPS))
                        ciou = iou - (rho2 / c2 + v * alpha)
                        box_s += jnp.sum((1.0 - ciou) * mf)
                        corr_s += jnp.sum(psc[4:5, :]
                                          * jnp.clip(ciou, 0.0) * mf)
                        nv_s += jnp.sum(mf)
                        xl = psc[5:85, :]
                        bce = (jnp.maximum(xl, 0.0) - xl * oneh[:, sl]
                               + jnp.log1p(jnp.exp(-jnp.abs(xl))))
                        cls_s += jnp.sum(bce * mf)
            acc_ref[3 + lvl] = box_s
            acc_ref[6 + lvl] = corr_s
            acc_ref[9 + lvl] = cls_s
            acc_ref[12 + lvl] = nv_s

    @pl.when(g == pl.num_programs(0) - 1)
    def _final():
        lbox = 0.0
        lobj = 0.0
        lcls = 0.0
        for lvl, (H, W) in enumerate(_GRIDS):
            denom = jnp.maximum(acc_ref[12 + lvl], 1.0)
            lbox += acc_ref[3 + lvl] / denom
            lcls += acc_ref[9 + lvl] / (denom * 80.0)
            lobj += ((acc_ref[lvl] - acc_ref[6 + lvl])
                     / (_B * 3 * H * W)) * _BAL[lvl]
        lbox = lbox * 3.54
        lobj = lobj * 64.3
        lcls = lcls * 37.4
        loss = lbox + lobj + lcls
        out_ref[0] = loss
        out_ref[1] = lbox
        out_ref[2] = lobj
        out_ref[3] = lcls


def _tc_main(preds_0, preds_1, preds_2, ps0, ps1, ps2, tt, img):
    grid = (_B * 3,)

    def pmap(l):
        return lambda g: (g // 3, 85 * (g % 3) + 4, 0, 0)

    out = pl.pallas_call(
        _tc_body,
        grid=grid,
        in_specs=[
            pl.BlockSpec((1, 1, 80, 80), pmap(0)),
            pl.BlockSpec((1, 1, 40, 40), pmap(1)),
            pl.BlockSpec((1, 1, 20, 20), pmap(2)),
            pl.BlockSpec((30, 85, 128), lambda g: (0, 0, 0)),
            pl.BlockSpec((30, 85, 128), lambda g: (0, 0, 0)),
            pl.BlockSpec((30, 85, 128), lambda g: (0, 0, 0)),
            pl.BlockSpec((6, _N), lambda g: (0, 0)),
            pl.BlockSpec(memory_space=pltpu.SMEM),
        ],
        out_specs=pl.BlockSpec(memory_space=pltpu.SMEM),
        out_shape=jax.ShapeDtypeStruct((4,), jnp.float32),
        scratch_shapes=[pltpu.SMEM((16,), jnp.float32)],
        interpret=_INTERPRET,
    )(preds_0, preds_1, preds_2, ps0, ps1, ps2, tt, img)
    return out


def _sc_gather_body(p0_hbm, p1_hbm, p2_hbm, nb_hbm, cx_hbm, cy_hbm, img_hbm,
                    out0, out1, out2, nbv, cxv, cyv, imgv, idxb, gbuf, sem):
    nc = 2
    w = lax.axis_index("s") * nc + lax.axis_index("c")

    @pl.when(w < 30)
    def _work():
        combo = w // 2
        a = combo // 5
        o = combo % 5
        t0 = (w % 2) * 128
        pltpu.sync_copy(nb_hbm.at[pl.ds(t0, 128)], nbv)
        pltpu.sync_copy(cx_hbm.at[pl.ds(t0, 128)], cxv)
        pltpu.sync_copy(cy_hbm.at[pl.ds(t0, 128)], cyv)
        pltpu.sync_copy(img_hbm, imgv)
        imgw = imgv[pl.ds(0, 16)]
        imgh = imgv[pl.ds(16, 16)]
        xo = jnp.where(o == 1, 1.0, 0.0) + jnp.where(o == 3, -1.0, 0.0)
        yo = jnp.where(o == 2, 1.0, 0.0) + jnp.where(o == 4, -1.0, 0.0)
        lane = lax.iota(jnp.int32, 16)
        for lvl, (H, W) in enumerate(_GRIDS):
            p_hbm = (p0_hbm, p1_hbm, p2_hbm)[lvl]
            out = (out0, out1, out2)[lvl]
            hw = H * W
            bases = []
            for gg in range(8):
                s = pl.ds(gg * 16, 16)
                gx = cxv[s] / (imgw / float(W)) - xo
                gy = cyv[s] / (imgh / float(H)) - yo
                gi = jnp.clip(gx.astype(jnp.int32), 0, W - 1)
                gj = jnp.clip(gy.astype(jnp.int32), 0, H - 1)
                b = jnp.clip(nbv[s].astype(jnp.int32), 0, _B - 1)
                bases.append(((b * 255 + 85 * a) * H + gj) * W + gi)

            def fill(c, _):
                coff = jnp.full((16,), c * hw, dtype=jnp.int32)
                for gg in range(8):
                    idxb[pl.ds(c * 128 + gg * 16, 16)] = bases[gg] + coff
                return _

            lax.fori_loop(0, 85, fill, 0, unroll=False)

            def fire(c, _):
                pltpu.async_copy(p_hbm.at[idxb.at[pl.ds(c * 128, 128)]],
                                 gbuf.at[c], sem)
                return _

            lax.fori_loop(0, 85, fire, 0, unroll=False)
            pltpu.make_async_copy(out.at[0], gbuf, sem).wait()
            pltpu.sync_copy(gbuf, out.at[w])


@functools.cache
def _sc_gather_fn():
    mesh = plsc.VectorSubcoreMesh(core_axis_name="c", subcore_axis_name="s")
    return pl.kernel(
        _sc_gather_body,
        mesh=mesh,
        out_type=[jax.ShapeDtypeStruct((30, 85, 128), jnp.float32)] * 3,
        scratch_types=[
            pltpu.VMEM((128,), jnp.float32),
            pltpu.VMEM((128,), jnp.float32),
            pltpu.VMEM((128,), jnp.float32),
            pltpu.VMEM((32,), jnp.float32),
            pltpu.VMEM((85 * 128,), jnp.int32),
            pltpu.VMEM((85, 128), jnp.float32),
            pltpu.SemaphoreType.DMA,
        ],
    )


def kernel(preds_0, preds_1, preds_2, targets, image_size):
    tt = targets[0].T.astype(jnp.float32)
    img = image_size.reshape(1, 2).astype(jnp.float32)
    img32 = jnp.concatenate([jnp.full((16,), image_size[0], jnp.float32),
                             jnp.full((16,), image_size[1], jnp.float32)])
    ps0, ps1, ps2 = _sc_gather_fn()(
        preds_0.reshape(-1), preds_1.reshape(-1), preds_2.reshape(-1),
        tt[0], tt[2], tt[3], img32)
    out = _tc_main(preds_0, preds_1, preds_2, ps0, ps1, ps2, tt, img)
    return (out[0:1], out[1:2], out[2:3], out[3:4])
